# baseline (speedup 1.0000x reference)
import jax
import jax.numpy as jnp
from jax.experimental import pallas as pl
from jax.experimental.pallas import tpu as pltpu

_SIZE_DICT = 8192
_DIM = 32
_TEMP = 0.5

_N = 16384
_B = 128
_NB = _N // _B
_BS = 16


def _to_i32(v):
    v &= 0xFFFFFFFF
    return v - (1 << 32) if v >= (1 << 31) else v


_KS = (0, 42, _to_i32(0 ^ 42 ^ 0x1BD11BDA))
_ROT_GROUPS = ((13, 15, 26, 6), (17, 29, 16, 24))


def _rotl(x, d):
    return jax.lax.shift_left(x, jnp.int32(d)) | jax.lax.shift_right_logical(
        x, jnp.int32(32 - d))


def _threefry_bits(flat_idx):
    x0 = jnp.zeros_like(flat_idx) + jnp.int32(_KS[0])
    x1 = flat_idx + jnp.int32(_KS[1])
    for i in range(5):
        for r in _ROT_GROUPS[i % 2]:
            x0 = x0 + x1
            x1 = _rotl(x1, r)
            x1 = x0 ^ x1
        x0 = x0 + jnp.int32(_KS[(i + 1) % 3])
        x1 = x1 + jnp.int32(_to_i32(_KS[(i + 2) % 3] + i + 1))
    return x0 ^ x1


def _vq_kernel(z_ref, cb_ref, var_ref, zq_ref, idx_ref, avg_ref, loss_ref,
               perp_ref, kd_acc, sq_acc, flat0_ref):
    i = pl.program_id(0)
    B, K = _B, _SIZE_DICT

    @pl.when(i == 0)
    def _init():
        avg_ref[...] = jnp.zeros_like(avg_ref)
        kd_acc[...] = jnp.zeros((1, 1), jnp.float32)
        sq_acc[...] = jnp.zeros((1, 1), jnp.float32)
        row = jax.lax.broadcasted_iota(jnp.int32, (B, K), 0)
        cols = jax.lax.broadcasted_iota(jnp.int32, (B, K), 1)
        flat0_ref[...] = row * jnp.int32(K) + cols

    z = z_ref[...]
    cb = cb_ref[...]
    var = var_ref[...]
    w = 0.5 / jnp.maximum(var, 1e-10)

    z2 = jnp.sum(z * z, axis=1, keepdims=True)
    c2 = jnp.sum(cb * cb, axis=1)[None, :]
    zc = jax.lax.dot_general(
        z, cb, dimension_numbers=(((1,), (1,)), ((), ())),
        preferred_element_type=jnp.float32,
        precision=jax.lax.Precision.DEFAULT)
    logit = -(w * (z2 + c2 - 2.0 * zc))

    m = jnp.max(logit, axis=1, keepdims=True)
    flat0 = flat0_ref[...]
    col = jax.lax.bitwise_and(flat0, jnp.int32(K - 1))
    idx_ref[...] = jnp.min(
        jnp.where(logit == m, col, jnp.int32(K)), axis=1, keepdims=True)

    shifted = logit - m
    p_un = jnp.exp(shifted)
    s = jnp.sum(p_un, axis=1, keepdims=True)
    inv_s = 1.0 / s
    t = jnp.sum(p_un * shifted, axis=1, keepdims=True)
    kd_acc[...] += jnp.sum(t * inv_s - jnp.log(s), axis=0, keepdims=True)
    avg_ref[...] += jax.lax.dot_general(
        inv_s * jnp.float32(1.0 / _N), p_un,
        dimension_numbers=(((0,), (0,)), ((), ())),
        preferred_element_type=jnp.float32,
        precision=jax.lax.Precision.HIGHEST)

    bits = _threefry_bits(flat0 + jnp.int32(i * B * K))
    u_bits = jax.lax.shift_right_logical(bits, jnp.int32(9)) | jnp.int32(
        0x3F800000)
    u = jax.lax.bitcast_convert_type(u_bits, jnp.float32) - 1.0
    ell = jnp.float32(1e-10) - jnp.log(u + 1e-10)
    q = p_un / ell
    e_un = q * q
    s2 = jnp.sum(e_un, axis=1, keepdims=True)
    enc = e_un * (1.0 / s2)
    zq = jax.lax.dot_general(
        enc, cb, dimension_numbers=(((1,), (0,)), ((), ())),
        preferred_element_type=jnp.float32,
        precision=jax.lax.Precision.DEFAULT)
    zq_ref[...] = zq
    sq = (z - zq) ** 2
    sq_acc[...] += jnp.sum(
        jnp.sum(sq, axis=1, keepdims=True), axis=0, keepdims=True)

    @pl.when(i == _NB - 1)
    def _fin():
        prec = 1.0 / jnp.maximum(var, 1e-10)
        kd = kd_acc[...] / jnp.float32(_BS)
        kc = sq_acc[...] * (0.5 * prec) / jnp.float32(_BS)
        loss_ref[...] = kd + kc
        avg = avg_ref[...]
        perp_ref[...] = jnp.exp(
            -jnp.sum(avg * jnp.log(avg + 1e-7), axis=1, keepdims=True))


def kernel(z_pos, var_q_pos, codebook, flg_train):
    bs, dim_z, width, height = z_pos.shape
    z_flat = jnp.transpose(z_pos, (0, 2, 3, 1)).reshape(-1, _DIM)
    var2d = jnp.reshape(var_q_pos, (1, 1))

    zq_flat, idx, avg, loss, perp = pl.pallas_call(
        _vq_kernel,
        grid=(_NB,),
        in_specs=[
            pl.BlockSpec((_B, _DIM), lambda i: (i, 0)),
            pl.BlockSpec((_SIZE_DICT, _DIM), lambda i: (0, 0)),
            pl.BlockSpec((1, 1), lambda i: (0, 0)),
        ],
        out_specs=[
            pl.BlockSpec((_B, _DIM), lambda i: (i, 0)),
            pl.BlockSpec((_B, 1), lambda i: (i, 0)),
            pl.BlockSpec((1, _SIZE_DICT), lambda i: (0, 0)),
            pl.BlockSpec((1, 1), lambda i: (0, 0)),
            pl.BlockSpec((1, 1), lambda i: (0, 0)),
        ],
        out_shape=[
            jax.ShapeDtypeStruct((_N, _DIM), jnp.float32),
            jax.ShapeDtypeStruct((_N, 1), jnp.int32),
            jax.ShapeDtypeStruct((1, _SIZE_DICT), jnp.float32),
            jax.ShapeDtypeStruct((1, 1), jnp.float32),
            jax.ShapeDtypeStruct((1, 1), jnp.float32),
        ],
        scratch_shapes=[
            pltpu.VMEM((1, 1), jnp.float32),
            pltpu.VMEM((1, 1), jnp.float32),
            pltpu.VMEM((_B, _SIZE_DICT), jnp.int32),
        ],
    )(z_flat, codebook, var2d)

    z_to_decoder = zq_flat.reshape(bs, width, height, dim_z).transpose(
        0, 3, 1, 2)
    idx_out = idx.reshape(bs, width, height)
    return (loss[0, 0], z_to_decoder, perp[0, 0], avg[0], idx_out)


# speedup vs baseline: 1.1072x; 1.1072x over previous
import jax
import jax.numpy as jnp
from jax.experimental import pallas as pl
from jax.experimental.pallas import tpu as pltpu

_SIZE_DICT = 8192
_DIM = 32
_TEMP = 0.5

_N = 16384
_B = 128
_NB = _N // _B
_BS = 16


def _to_i32(v):
    v &= 0xFFFFFFFF
    return v - (1 << 32) if v >= (1 << 31) else v


_KS = (0, 42, _to_i32(0 ^ 42 ^ 0x1BD11BDA))
_ROT_GROUPS = ((13, 15, 26, 6), (17, 29, 16, 24))


def _rotl(x, d):
    return jax.lax.shift_left(x, jnp.int32(d)) | jax.lax.shift_right_logical(
        x, jnp.int32(32 - d))


def _threefry_bits(flat_idx):
    x1 = flat_idx + jnp.int32(_KS[1])
    x0 = x1
    x1 = x0 ^ _rotl(x1, _ROT_GROUPS[0][0])
    first = True
    for i in range(5):
        for r in _ROT_GROUPS[i % 2]:
            if first:
                first = False
                continue
            x0 = x0 + x1
            x1 = _rotl(x1, r)
            x1 = x0 ^ x1
        if _KS[(i + 1) % 3]:
            x0 = x0 + jnp.int32(_KS[(i + 1) % 3])
        x1 = x1 + jnp.int32(_to_i32(_KS[(i + 2) % 3] + i + 1))
    return x0 ^ x1


def _vq_kernel(z_ref, cb_ref, var_ref, flat0_ref, zq_ref, idx_ref, avg_ref,
               loss_ref, perp_ref, kd_acc, sq_acc):
    i = pl.program_id(0)
    B, K = _B, _SIZE_DICT

    @pl.when(i == 0)
    def _init():
        avg_ref[...] = jnp.zeros_like(avg_ref)
        kd_acc[...] = jnp.zeros((1, 1), jnp.float32)
        sq_acc[...] = jnp.zeros((1, 1), jnp.float32)

    z = z_ref[...]
    cb = cb_ref[...]
    var = var_ref[...]
    w = 0.5 / jnp.maximum(var, 1e-10)

    z2 = jnp.sum(z * z, axis=1, keepdims=True)
    c2 = jnp.sum(cb * cb, axis=1)[None, :]
    zc = jax.lax.dot_general(
        z, cb, dimension_numbers=(((1,), (1,)), ((), ())),
        preferred_element_type=jnp.float32,
        precision=jax.lax.Precision.DEFAULT)
    logit = -(w * (z2 + c2 - 2.0 * zc))

    m = jnp.max(logit, axis=1, keepdims=True)
    flat0 = flat0_ref[...]
    col = jax.lax.bitwise_and(flat0, jnp.int32(K - 1))
    idx_ref[...] = jnp.min(
        jnp.where(logit == m, col, jnp.int32(K)), axis=1, keepdims=True)

    shifted = logit - m
    p_un = jnp.exp(shifted)
    s = jnp.sum(p_un, axis=1, keepdims=True)
    inv_s = 1.0 / s
    t = jnp.sum(p_un * shifted, axis=1, keepdims=True)
    kd_acc[...] += jnp.sum(t * inv_s - jnp.log(s), axis=0, keepdims=True)
    avg_ref[...] += jax.lax.dot_general(
        inv_s * jnp.float32(1.0 / _N), p_un,
        dimension_numbers=(((0,), (0,)), ((), ())),
        preferred_element_type=jnp.float32,
        precision=jax.lax.Precision.DEFAULT)

    bits = _threefry_bits(flat0 + jnp.int32(i * B * K))
    u_bits = jax.lax.shift_right_logical(bits, jnp.int32(9)) | jnp.int32(
        0x3F800000)
    u = jax.lax.bitcast_convert_type(u_bits, jnp.float32) - 1.0
    ell = jnp.float32(1e-10) - jnp.log(u + 1e-10)
    q = p_un / ell
    e_un = q * q
    s2 = jnp.sum(e_un, axis=1, keepdims=True)
    enc = e_un * (1.0 / s2)
    zq = jax.lax.dot_general(
        enc, cb, dimension_numbers=(((1,), (0,)), ((), ())),
        preferred_element_type=jnp.float32,
        precision=jax.lax.Precision.DEFAULT)
    zq_ref[...] = zq
    sq = (z - zq) ** 2
    sq_acc[...] += jnp.sum(
        jnp.sum(sq, axis=1, keepdims=True), axis=0, keepdims=True)

    @pl.when(i == _NB - 1)
    def _fin():
        prec = 1.0 / jnp.maximum(var, 1e-10)
        kd = kd_acc[...] / jnp.float32(_BS)
        kc = sq_acc[...] * (0.5 * prec) / jnp.float32(_BS)
        loss_ref[...] = kd + kc
        avg = avg_ref[...]
        perp_ref[...] = jnp.exp(
            -jnp.sum(avg * jnp.log(avg + 1e-7), axis=1, keepdims=True))


def kernel(z_pos, var_q_pos, codebook, flg_train):
    bs, dim_z, width, height = z_pos.shape
    z_flat = jnp.transpose(z_pos, (0, 2, 3, 1)).reshape(-1, _DIM)
    var2d = jnp.reshape(var_q_pos, (1, 1))
    flat0 = jnp.arange(_B * _SIZE_DICT, dtype=jnp.int32).reshape(
        _B, _SIZE_DICT)

    zq_flat, idx, avg, loss, perp = pl.pallas_call(
        _vq_kernel,
        grid=(_NB,),
        in_specs=[
            pl.BlockSpec((_B, _DIM), lambda i: (i, 0)),
            pl.BlockSpec((_SIZE_DICT, _DIM), lambda i: (0, 0)),
            pl.BlockSpec((1, 1), lambda i: (0, 0)),
            pl.BlockSpec((_B, _SIZE_DICT), lambda i: (0, 0)),
        ],
        out_specs=[
            pl.BlockSpec((_B, _DIM), lambda i: (i, 0)),
            pl.BlockSpec((_B, 1), lambda i: (i, 0)),
            pl.BlockSpec((1, _SIZE_DICT), lambda i: (0, 0)),
            pl.BlockSpec((1, 1), lambda i: (0, 0)),
            pl.BlockSpec((1, 1), lambda i: (0, 0)),
        ],
        out_shape=[
            jax.ShapeDtypeStruct((_N, _DIM), jnp.float32),
            jax.ShapeDtypeStruct((_N, 1), jnp.int32),
            jax.ShapeDtypeStruct((1, _SIZE_DICT), jnp.float32),
            jax.ShapeDtypeStruct((1, 1), jnp.float32),
            jax.ShapeDtypeStruct((1, 1), jnp.float32),
        ],
        scratch_shapes=[
            pltpu.VMEM((1, 1), jnp.float32),
            pltpu.VMEM((1, 1), jnp.float32),
        ],
    )(z_flat, codebook, var2d, flat0)

    z_to_decoder = zq_flat.reshape(bs, width, height, dim_z).transpose(
        0, 3, 1, 2)
    idx_out = idx.reshape(bs, width, height)
    return (loss[0, 0], z_to_decoder, perp[0, 0], avg[0], idx_out)


# speedup vs baseline: 1.1562x; 1.0442x over previous
import jax
import jax.numpy as jnp
from jax.experimental import pallas as pl
from jax.experimental.pallas import tpu as pltpu

_SIZE_DICT = 8192
_DIM = 32
_TEMP = 0.5

_N = 16384
_B = 256
_NB = _N // _B
_BS = 16


def _to_i32(v):
    v &= 0xFFFFFFFF
    return v - (1 << 32) if v >= (1 << 31) else v


_KS = (0, 42, _to_i32(0 ^ 42 ^ 0x1BD11BDA))
_ROT_GROUPS = ((13, 15, 26, 6), (17, 29, 16, 24))


def _rotl(x, d):
    return jax.lax.shift_left(x, jnp.int32(d)) | jax.lax.shift_right_logical(
        x, jnp.int32(32 - d))


def _threefry_bits(flat_idx):
    x1 = flat_idx + jnp.int32(_KS[1])
    x0 = x1
    x1 = x0 ^ _rotl(x1, _ROT_GROUPS[0][0])
    first = True
    for i in range(5):
        for r in _ROT_GROUPS[i % 2]:
            if first:
                first = False
                continue
            x0 = x0 + x1
            x1 = _rotl(x1, r)
            x1 = x0 ^ x1
        if _KS[(i + 1) % 3]:
            x0 = x0 + jnp.int32(_KS[(i + 1) % 3])
        x1 = x1 + jnp.int32(_to_i32(_KS[(i + 2) % 3] + i + 1))
    return x0 ^ x1


def _vq_kernel(z_ref, cb_ref, var_ref, zq_ref, idx_ref, avg_ref,
               loss_ref, perp_ref, kd_acc, sq_acc):
    i = pl.program_id(0)
    B, K = _B, _SIZE_DICT

    @pl.when(i == 0)
    def _init():
        avg_ref[...] = jnp.zeros_like(avg_ref)
        kd_acc[...] = jnp.zeros((1, 1), jnp.float32)
        sq_acc[...] = jnp.zeros((1, 1), jnp.float32)

    z = z_ref[...]
    cb = cb_ref[...]
    var = var_ref[...]
    w = 0.5 / jnp.maximum(var, 1e-10)

    z2 = jnp.sum(z * z, axis=1, keepdims=True)
    c2 = jnp.sum(cb * cb, axis=1)[None, :]
    zc = jax.lax.dot_general(
        z, cb, dimension_numbers=(((1,), (1,)), ((), ())),
        preferred_element_type=jnp.float32,
        precision=jax.lax.Precision.DEFAULT)
    logit = -(w * (z2 + c2 - 2.0 * zc))

    m = jnp.max(logit, axis=1, keepdims=True)
    col = jax.lax.broadcasted_iota(jnp.int32, (B, K), 1)
    idx_ref[...] = jnp.min(
        jnp.where(logit == m, col, jnp.int32(K)), axis=1, keepdims=True)

    shifted = logit - m
    p_un = jnp.exp(shifted)
    s = jnp.sum(p_un, axis=1, keepdims=True)
    inv_s = 1.0 / s
    t = jnp.sum(p_un * shifted, axis=1, keepdims=True)
    kd_acc[...] += jnp.sum(t * inv_s - jnp.log(s), axis=0, keepdims=True)
    avg_ref[...] += jax.lax.dot_general(
        inv_s * jnp.float32(1.0 / _N), p_un,
        dimension_numbers=(((0,), (0,)), ((), ())),
        preferred_element_type=jnp.float32,
        precision=jax.lax.Precision.DEFAULT)

    row = jax.lax.broadcasted_iota(jnp.int32, (B, K), 0)
    flat = jnp.int32(i * B * K) + row * jnp.int32(K) + col
    bits = _threefry_bits(flat)
    u_bits = jax.lax.shift_right_logical(bits, jnp.int32(9)) | jnp.int32(
        0x3F800000)
    u = jax.lax.bitcast_convert_type(u_bits, jnp.float32) - 1.0
    g = -jnp.log(-jnp.log(u + 1e-10) + 1e-10)
    y = (logit + g) * jnp.float32(1.0 / _TEMP)
    m2 = jnp.max(y, axis=1, keepdims=True)
    e_un = jnp.exp(y - m2)
    s2 = jnp.sum(e_un, axis=1, keepdims=True)
    enc = e_un * (1.0 / s2)
    zq = jax.lax.dot_general(
        enc, cb, dimension_numbers=(((1,), (0,)), ((), ())),
        preferred_element_type=jnp.float32,
        precision=jax.lax.Precision.DEFAULT)
    zq_ref[...] = zq
    sq = (z - zq) ** 2
    sq_acc[...] += jnp.sum(
        jnp.sum(sq, axis=1, keepdims=True), axis=0, keepdims=True)

    @pl.when(i == _NB - 1)
    def _fin():
        prec = 1.0 / jnp.maximum(var, 1e-10)
        kd = kd_acc[...] / jnp.float32(_BS)
        kc = sq_acc[...] * (0.5 * prec) / jnp.float32(_BS)
        loss_ref[...] = kd + kc
        avg = avg_ref[...]
        perp_ref[...] = jnp.exp(
            -jnp.sum(avg * jnp.log(avg + 1e-7), axis=1, keepdims=True))


def kernel(z_pos, var_q_pos, codebook, flg_train):
    bs, dim_z, width, height = z_pos.shape
    z_flat = jnp.transpose(z_pos, (0, 2, 3, 1)).reshape(-1, _DIM)
    var2d = jnp.reshape(var_q_pos, (1, 1))

    zq_flat, idx, avg, loss, perp = pl.pallas_call(
        _vq_kernel,
        grid=(_NB,),
        in_specs=[
            pl.BlockSpec((_B, _DIM), lambda i: (i, 0)),
            pl.BlockSpec((_SIZE_DICT, _DIM), lambda i: (0, 0)),
            pl.BlockSpec((1, 1), lambda i: (0, 0)),
        ],
        out_specs=[
            pl.BlockSpec((_B, _DIM), lambda i: (i, 0)),
            pl.BlockSpec((_B, 1), lambda i: (i, 0)),
            pl.BlockSpec((1, _SIZE_DICT), lambda i: (0, 0)),
            pl.BlockSpec((1, 1), lambda i: (0, 0)),
            pl.BlockSpec((1, 1), lambda i: (0, 0)),
        ],
        out_shape=[
            jax.ShapeDtypeStruct((_N, _DIM), jnp.float32),
            jax.ShapeDtypeStruct((_N, 1), jnp.int32),
            jax.ShapeDtypeStruct((1, _SIZE_DICT), jnp.float32),
            jax.ShapeDtypeStruct((1, 1), jnp.float32),
            jax.ShapeDtypeStruct((1, 1), jnp.float32),
        ],
        scratch_shapes=[
            pltpu.VMEM((1, 1), jnp.float32),
            pltpu.VMEM((1, 1), jnp.float32),
        ],
    )(z_flat, codebook, var2d)

    z_to_decoder = zq_flat.reshape(bs, width, height, dim_z).transpose(
        0, 3, 1, 2)
    idx_out = idx.reshape(bs, width, height)
    return (loss[0, 0], z_to_decoder, perp[0, 0], avg[0], idx_out)


# speedup vs baseline: 1.3090x; 1.1322x over previous
import jax
import jax.numpy as jnp
from jax.experimental import pallas as pl
from jax.experimental.pallas import tpu as pltpu

_SIZE_DICT = 8192
_DIM = 32
_TEMP = 0.5

_N = 16384
_B = 128
_NB = _N // _B
_BS = 16


def _to_i32(v):
    v &= 0xFFFFFFFF
    return v - (1 << 32) if v >= (1 << 31) else v


_KS = (0, 42, _to_i32(0 ^ 42 ^ 0x1BD11BDA))
_ROT_GROUPS = ((13, 15, 26, 6), (17, 29, 16, 24))


def _rotl(x, d):
    return jax.lax.shift_left(x, jnp.int32(d)) | jax.lax.shift_right_logical(
        x, jnp.int32(32 - d))


def _threefry_bits(flat_idx):
    x1 = flat_idx + jnp.int32(_KS[1])
    x0 = x1
    x1 = x0 ^ _rotl(x1, _ROT_GROUPS[0][0])
    first = True
    for i in range(5):
        for r in _ROT_GROUPS[i % 2]:
            if first:
                first = False
                continue
            x0 = x0 + x1
            x1 = _rotl(x1, r)
            x1 = x0 ^ x1
        if _KS[(i + 1) % 3]:
            x0 = x0 + jnp.int32(_KS[(i + 1) % 3])
        x1 = x1 + jnp.int32(_to_i32(_KS[(i + 2) % 3] + i + 1))
    return x0 ^ x1


def _vq_kernel(z_ref, cb_ref, var_ref, zq_ref, idx_ref, avg_ref,
               loss_ref, perp_ref, kd_acc, sq_acc):
    i = pl.program_id(0)
    B, K = _B, _SIZE_DICT

    @pl.when(i == 0)
    def _init():
        avg_ref[...] = jnp.zeros_like(avg_ref)
        kd_acc[...] = jnp.zeros((1, 1), jnp.float32)
        sq_acc[...] = jnp.zeros((1, 1), jnp.float32)

    z = z_ref[...]
    cb = cb_ref[...]
    var = var_ref[...]
    w = 0.5 / jnp.maximum(var, 1e-10)

    z2 = jnp.sum(z * z, axis=1, keepdims=True)
    c2 = jnp.sum(cb * cb, axis=1)[None, :]
    zc = jax.lax.dot_general(
        z, cb, dimension_numbers=(((1,), (1,)), ((), ())),
        preferred_element_type=jnp.float32,
        precision=jax.lax.Precision.DEFAULT)
    logit = -(w * (z2 + c2 - 2.0 * zc))

    m = jnp.max(logit, axis=1, keepdims=True)
    col = jax.lax.broadcasted_iota(jnp.int32, (B, K), 1)
    idx_ref[...] = jnp.min(
        jnp.where(logit == m, col, jnp.int32(K)), axis=1, keepdims=True)

    shifted = logit - m
    p_un = jnp.exp(shifted)
    s = jnp.sum(p_un, axis=1, keepdims=True)
    inv_s = 1.0 / s
    t = jnp.sum(p_un * shifted, axis=1, keepdims=True)
    kd_acc[...] += jnp.sum(t * inv_s - jnp.log(s), axis=0, keepdims=True)
    avg_ref[...] += jax.lax.dot_general(
        inv_s * jnp.float32(1.0 / _N), p_un,
        dimension_numbers=(((0,), (0,)), ((), ())),
        preferred_element_type=jnp.float32,
        precision=jax.lax.Precision.DEFAULT)

    row = jax.lax.broadcasted_iota(jnp.int32, (B, K), 0)
    flat = jnp.int32(i * B * K) + row * jnp.int32(K) + col
    bits = _threefry_bits(flat)
    u_bits = jax.lax.shift_right_logical(bits, jnp.int32(9)) | jnp.int32(
        0x3F800000)
    u = jax.lax.bitcast_convert_type(u_bits, jnp.float32) - 1.0
    g = -jnp.log(-jnp.log(u + 1e-10) + 1e-10)
    y = (logit + g) * jnp.float32(1.0 / _TEMP)
    m2 = jnp.max(y, axis=1, keepdims=True)
    e_un = jnp.exp(y - m2)
    s2 = jnp.sum(e_un, axis=1, keepdims=True)
    enc = e_un * (1.0 / s2)
    zq = jax.lax.dot_general(
        enc, cb, dimension_numbers=(((1,), (0,)), ((), ())),
        preferred_element_type=jnp.float32,
        precision=jax.lax.Precision.DEFAULT)
    zq_ref[...] = zq
    sq = (z - zq) ** 2
    sq_acc[...] += jnp.sum(
        jnp.sum(sq, axis=1, keepdims=True), axis=0, keepdims=True)

    @pl.when(i == _NB - 1)
    def _fin():
        prec = 1.0 / jnp.maximum(var, 1e-10)
        kd = kd_acc[...] / jnp.float32(_BS)
        kc = sq_acc[...] * (0.5 * prec) / jnp.float32(_BS)
        loss_ref[...] = kd + kc
        avg = avg_ref[...]
        perp_ref[...] = jnp.exp(
            -jnp.sum(avg * jnp.log(avg + 1e-7), axis=1, keepdims=True))


def kernel(z_pos, var_q_pos, codebook, flg_train):
    bs, dim_z, width, height = z_pos.shape
    z_flat = jnp.transpose(z_pos, (0, 2, 3, 1)).reshape(-1, _DIM)
    var2d = jnp.reshape(var_q_pos, (1, 1))

    zq_flat, idx, avg, loss, perp = pl.pallas_call(
        _vq_kernel,
        grid=(_NB,),
        in_specs=[
            pl.BlockSpec((_B, _DIM), lambda i: (i, 0)),
            pl.BlockSpec((_SIZE_DICT, _DIM), lambda i: (0, 0)),
            pl.BlockSpec((1, 1), lambda i: (0, 0)),
        ],
        out_specs=[
            pl.BlockSpec((_B, _DIM), lambda i: (i, 0)),
            pl.BlockSpec((_B, 1), lambda i: (i, 0)),
            pl.BlockSpec((1, _SIZE_DICT), lambda i: (0, 0)),
            pl.BlockSpec((1, 1), lambda i: (0, 0)),
            pl.BlockSpec((1, 1), lambda i: (0, 0)),
        ],
        out_shape=[
            jax.ShapeDtypeStruct((_N, _DIM), jnp.float32),
            jax.ShapeDtypeStruct((_N, 1), jnp.int32),
            jax.ShapeDtypeStruct((1, _SIZE_DICT), jnp.float32),
            jax.ShapeDtypeStruct((1, 1), jnp.float32),
            jax.ShapeDtypeStruct((1, 1), jnp.float32),
        ],
        scratch_shapes=[
            pltpu.VMEM((1, 1), jnp.float32),
            pltpu.VMEM((1, 1), jnp.float32),
        ],
    )(z_flat, codebook, var2d)

    z_to_decoder = zq_flat.reshape(bs, width, height, dim_z).transpose(
        0, 3, 1, 2)
    idx_out = idx.reshape(bs, width, height)
    return (loss[0, 0], z_to_decoder, perp[0, 0], avg[0], idx_out)
